# baseline (speedup 1.0000x reference)
import functools

import jax
import jax.numpy as jnp
from jax import lax
from jax.experimental import pallas as pl
from jax.experimental.pallas import tpu as pltpu
from jax.experimental.pallas import tpu_sc as plsc

NU = 40000
NI = 60000
NN = NU + NI
D = 32
H = 16
E = 1600000
NT = 16
B = 384
SUB = B // 128
C = 264
EP = NT * C * B
NR128 = EP // 128
R3 = 3
R6 = 6
RPT_A = 6256
RPT_L = NN - 15 * RPT_A
ZB = 128
CH = 125
UPT = NU // NT
IPT = NI // NT


def _edge_phase(c, s, table, rows_h, cols_h, w_h,
                colb, rowb, wb, gb, zb, acc, sem_ld, sem_g, sem_s, sz):
    chunk0 = s * C

    def start_loads(g, b6):
        crow = (chunk0 + g) * SUB
        pltpu.async_copy(rows_h.at[pl.ds(crow, SUB)],
                         rowb.at[pl.ds(b6 * SUB, SUB)], sem_ld[b6])
        pltpu.async_copy(cols_h.at[pl.ds(crow, SUB)],
                         colb.at[pl.ds(b6 * SUB, SUB)], sem_ld[b6])
        pltpu.async_copy(w_h.at[pl.ds(crow, SUB)],
                         wb.at[pl.ds(b6 * SUB, SUB)], sem_ld[b6])

    def wait_loads(b6):
        pltpu.make_async_copy(rows_h.at[pl.ds(0, SUB)],
                              rowb.at[pl.ds(b6 * SUB, SUB)], sem_ld[b6]).wait()
        pltpu.make_async_copy(cols_h.at[pl.ds(0, SUB)],
                              colb.at[pl.ds(b6 * SUB, SUB)], sem_ld[b6]).wait()
        pltpu.make_async_copy(w_h.at[pl.ds(0, SUB)],
                              wb.at[pl.ds(b6 * SUB, SUB)], sem_ld[b6]).wait()

    def start_gathers(b3, b6):
        @pl.when(c == 0)
        def _():
            for k in range(SUB):
                pltpu.async_copy(table.at[0].at[colb.at[b6 * SUB + k]],
                                 gb.at[pl.ds(b3 * B + k * 128, 128)],
                                 sem_g[b3])

        @pl.when(c == 1)
        def _():
            for k in range(SUB):
                pltpu.async_copy(table.at[1].at[colb.at[b6 * SUB + k]],
                                 gb.at[pl.ds(b3 * B + k * 128, 128)],
                                 sem_g[b3])

    def wait_gathers(b3, b6):
        for k in range(SUB):
            pltpu.make_async_copy(table.at[0].at[colb.at[b6 * SUB + k]],
                                  gb.at[pl.ds(b3 * B + k * 128, 128)],
                                  sem_g[b3]).wait()

    def scale_scatter(b3, b6):
        for k in range(SUB):
            base = b3 * B + k * 128
            wrow = b6 * SUB + k

            def sbody(j16, _, base=base, wrow=wrow):
                wvec = wb[wrow, pl.ds(j16 * 16, 16)]
                r0 = base + j16 * 16
                for l in range(16):
                    gb[r0 + l, :] = gb[r0 + l, :] * wvec[l]
                return 0

            lax.fori_loop(0, 8, sbody, 0)
            pltpu.async_copy(gb.at[pl.ds(base, 128)],
                             acc.at[rowb.at[wrow]], sem_s[b3], add=True)

    def wait_scatters(b3, b6):
        for k in range(SUB):
            pltpu.make_async_copy(gb.at[pl.ds(b3 * B + k * 128, 128)],
                                  acc.at[rowb.at[b6 * SUB + k]],
                                  sem_s[b3]).wait()

    def zb_body(i, _):
        zb[i, :] = jnp.zeros((H,), jnp.float32)
        return 0

    lax.fori_loop(0, ZB, zb_body, 0)

    zbase = s * RPT_A
    nfull = RPT_A // ZB
    for q in range(nfull):
        pltpu.async_copy(zb, acc.at[pl.ds(zbase + q * ZB, ZB)], sz)
    for q in range(nfull):
        pltpu.make_async_copy(zb, acc.at[pl.ds(zbase + q * ZB, ZB)], sz).wait()

    @pl.when(s < NT - 1)
    def _():
        tail = RPT_A - nfull * ZB
        pltpu.sync_copy(zb.at[pl.ds(0, tail)],
                        acc.at[pl.ds(zbase + nfull * ZB, tail)])

    @pl.when(s == NT - 1)
    def _():
        tail = RPT_L - nfull * ZB
        pltpu.sync_copy(zb.at[pl.ds(0, tail)],
                        acc.at[pl.ds(zbase + nfull * ZB, tail)])

    for b in range(3):
        start_loads(b, b)

    plsc.subcore_barrier()

    def group(tg, _):
        for u in range(6):
            g = tg * 6 + u

            @pl.when(g >= 2)
            def _(u=u):
                wait_scatters((u - 2) % 3, (u - 2) % 6)

            @pl.when(g + 3 < C)
            def _(g=g, u=u):
                start_loads(g + 3, (u + 3) % 6)

            wait_loads(u)
            start_gathers(u % 3, u)

            @pl.when(g >= 1)
            def _(u=u):
                wait_gathers((u - 1) % 3, (u - 1) % 6)
                scale_scatter((u - 1) % 3, (u - 1) % 6)
        return 0

    lax.fori_loop(0, C // 6, group, 0)

    wait_gathers((C - 1) % 3, (C - 1) % 6)
    scale_scatter((C - 1) % 3, (C - 1) % 6)
    wait_scatters((C - 2) % 3, (C - 2) % 6)
    wait_scatters((C - 1) % 3, (C - 1) % 6)

    plsc.subcore_barrier()

    return zbase


def _layer_body(table, rows_h, cols_h, w_h, out,
                colb, rowb, wb, gb, zb, acc,
                sl0, sl1, sl2, sl3, sl4, sl5,
                sg0, sg1, sg2, ss0, ss1, ss2, sz):
    c = lax.axis_index("c")
    s = lax.axis_index("s")
    zbase = _edge_phase(c, s, table, rows_h, cols_h, w_h,
                        colb, rowb, wb, gb, zb, acc,
                        (sl0, sl1, sl2, sl3, sl4, sl5),
                        (sg0, sg1, sg2), (ss0, ss1, ss2), sz)

    @pl.when(s < NT - 1)
    def _():
        pltpu.sync_copy(acc.at[pl.ds(zbase, RPT_A)],
                        out.at[c, pl.ds(zbase, RPT_A)])

    @pl.when(s == NT - 1)
    def _():
        pltpu.sync_copy(acc.at[pl.ds(zbase, RPT_L)],
                        out.at[c, pl.ds(zbase, RPT_L)])


def _final_body(table, rows_h, cols_h, w_h, e0h, e1h, e3out, users, items,
                colb, rowb, wb, gb, zb, acc,
                sl0, sl1, sl2, sl3, sl4, sl5,
                sg0, sg1, sg2, ss0, ss1, ss2, sz):
    c = lax.axis_index("c")
    s = lax.axis_index("s")
    zbase = _edge_phase(c, s, table, rows_h, cols_h, w_h,
                        colb, rowb, wb, gb, zb, acc,
                        (sl0, sl1, sl2, sl3, sl4, sl5),
                        (sg0, sg1, sg2), (ss0, ss1, ss2), sz)

    @pl.when(s < NT - 1)
    def _():
        pltpu.sync_copy(acc.at[pl.ds(zbase, RPT_A)],
                        e3out.at[c, pl.ds(zbase, RPT_A)])

    @pl.when(s == NT - 1)
    def _():
        pltpu.sync_copy(acc.at[pl.ds(zbase, RPT_L)],
                        e3out.at[c, pl.ds(zbase, RPT_L)])

    plsc.subcore_barrier()

    sem_m = (sl0, sl1)
    sem_w = (sl2, sl3)
    OFS = (0, 4 * CH)

    def m_load(racc, p):
        o = OFS[p]
        pltpu.async_copy(e0h.at[c, pl.ds(racc, CH)],
                         gb.at[pl.ds(o, CH)], sem_m[p])
        pltpu.async_copy(e1h.at[c, pl.ds(racc, CH)],
                         gb.at[pl.ds(o + CH, CH)], sem_m[p])
        pltpu.async_copy(table.at[c, pl.ds(racc, CH)],
                         gb.at[pl.ds(o + 2 * CH, CH)], sem_m[p])
        pltpu.async_copy(e3out.at[c, pl.ds(racc, CH)],
                         gb.at[pl.ds(o + 3 * CH, CH)], sem_m[p])

    def m_wait_load(p):
        o = OFS[p]
        pltpu.make_async_copy(e0h.at[0, pl.ds(0, CH)],
                              gb.at[pl.ds(o, CH)], sem_m[p]).wait()
        pltpu.make_async_copy(e1h.at[0, pl.ds(0, CH)],
                              gb.at[pl.ds(o + CH, CH)], sem_m[p]).wait()
        pltpu.make_async_copy(table.at[0, pl.ds(0, CH)],
                              gb.at[pl.ds(o + 2 * CH, CH)], sem_m[p]).wait()
        pltpu.make_async_copy(e3out.at[0, pl.ds(0, CH)],
                              gb.at[pl.ds(o + 3 * CH, CH)], sem_m[p]).wait()

    def m_compute(p):
        o = OFS[p]

        def mbody(i, _):
            gb[o + i, :] = (gb[o + i, :] + gb[o + CH + i, :] +
                            gb[o + 2 * CH + i, :] +
                            gb[o + 3 * CH + i, :]) * 0.25
            return 0

        lax.fori_loop(0, CH, mbody, 0)

    def m_store(out_ref, rout, p):
        pltpu.async_copy(gb.at[pl.ds(OFS[p], CH)],
                         out_ref.at[c, pl.ds(rout, CH)], sem_w[p])

    def m_wait_store(out_ref, p):
        pltpu.make_async_copy(gb.at[pl.ds(OFS[p], CH)],
                              out_ref.at[0, pl.ds(0, CH)],
                              sem_w[p]).wait()

    def run_half(out_ref, node_base, out_base, n2):
        m_load(node_base, 0)

        def mb(i, _):
            ra = node_base + (2 * i) * CH
            ro = out_base + (2 * i) * CH

            @pl.when(i > 0)
            def _():
                m_wait_store(out_ref, 1)

            m_load(ra + CH, 1)
            m_wait_load(0)
            m_compute(0)
            m_store(out_ref, ro, 0)
            m_wait_load(1)
            m_compute(1)
            m_store(out_ref, ro + CH, 1)
            m_wait_store(out_ref, 0)

            @pl.when(i + 1 < n2)
            def _():
                m_load(ra + 2 * CH, 0)
            return 0

        lax.fori_loop(0, n2, mb, 0)
        m_wait_store(out_ref, 1)

    run_half(users, s * UPT, s * UPT, UPT // (2 * CH))
    run_half(items, NU + s * IPT, s * IPT, IPT // (2 * CH))


_MESH = plsc.VectorSubcoreMesh(core_axis_name="c", subcore_axis_name="s")

_SCRATCH = [
    pltpu.VMEM((R6 * SUB, 128), jnp.int32),
    pltpu.VMEM((R6 * SUB, 128), jnp.int32),
    pltpu.VMEM((R6 * SUB, 128), jnp.float32),
    pltpu.VMEM((R3 * B, H), jnp.float32),
    pltpu.VMEM((ZB, H), jnp.float32),
    pltpu.VMEM_SHARED((NN, H), jnp.float32),
] + [pltpu.SemaphoreType.DMA] * 13

_layer = functools.partial(
    pl.kernel,
    out_type=jax.ShapeDtypeStruct((2, NN, H), jnp.float32),
    mesh=_MESH,
    compiler_params=pltpu.CompilerParams(use_tc_tiling_on_sc=False),
    scratch_types=_SCRATCH,
)(_layer_body)

_final = functools.partial(
    pl.kernel,
    out_type=(jax.ShapeDtypeStruct((2, NN, H), jnp.float32),
              jax.ShapeDtypeStruct((2, NU, H), jnp.float32),
              jax.ShapeDtypeStruct((2, NI, H), jnp.float32)),
    mesh=_MESH,
    compiler_params=pltpu.CompilerParams(use_tc_tiling_on_sc=False),
    scratch_types=_SCRATCH,
)(_final_body)


def kernel(edge_index, edge_weight, user_emb, item_emb):
    rows = edge_index[0].astype(jnp.int32)
    cols = edge_index[1].astype(jnp.int32)
    w = edge_weight.astype(jnp.float32)

    padr = NR128 - E // 128
    pidx = (jnp.arange(padr * 128, dtype=jnp.int32) % NN).reshape(padr, 128)
    rows_p = jnp.concatenate([rows.reshape(E // 128, 128), pidx], axis=0)
    cols_p = jnp.concatenate([cols.reshape(E // 128, 128), pidx], axis=0)
    w_p = jnp.pad(w.reshape(E // 128, 128), ((0, padr), (0, 0)))

    all_emb = jnp.concatenate([user_emb, item_emb], axis=0)
    e0 = jnp.stack([all_emb[:, :H], all_emb[:, H:]], axis=0)

    e1 = _layer(e0, rows_p, cols_p, w_p)
    e2 = _layer(e1, rows_p, cols_p, w_p)
    _, us, its = _final(e2, rows_p, cols_p, w_p, e0, e1)
    users = jnp.concatenate([us[0], us[1]], axis=1)
    items = jnp.concatenate([its[0], its[1]], axis=1)
    return (users, items)


# speedup vs baseline: 1.1453x; 1.1453x over previous
import functools

import jax
import jax.numpy as jnp
from jax import lax
from jax.experimental import pallas as pl
from jax.experimental.pallas import tpu as pltpu
from jax.experimental.pallas import tpu_sc as plsc

NU = 40000
NI = 60000
NN = NU + NI
D = 32
H = 16
E = 1600000
NT = 16
B = 384
SUB = B // 128
C = 264
EP = NT * C * B
NR128 = EP // 128
R3 = 3
R6 = 6
RPT_A = 6256
RPT_L = NN - 15 * RPT_A
ZB = 128
CH = 125
UPT = NU // NT
IPT = NI // NT


def _edge_phase(c, s, table, rows_h, cols_h, w_h,
                colb, rowb, wb, gb, zb, acc, sem_ld, sem_g, sem_s, sz):
    chunk0 = s * C

    def start_loads(g, b6):
        crow = (chunk0 + g) * SUB
        pltpu.async_copy(rows_h.at[pl.ds(crow, SUB)],
                         rowb.at[pl.ds(b6 * SUB, SUB)], sem_ld[b6])
        pltpu.async_copy(cols_h.at[pl.ds(crow, SUB)],
                         colb.at[pl.ds(b6 * SUB, SUB)], sem_ld[b6])
        pltpu.async_copy(w_h.at[pl.ds(crow, SUB)],
                         wb.at[pl.ds(b6 * SUB, SUB)], sem_ld[b6])

    def wait_loads(b6):
        pltpu.make_async_copy(rows_h.at[pl.ds(0, SUB)],
                              rowb.at[pl.ds(b6 * SUB, SUB)], sem_ld[b6]).wait()
        pltpu.make_async_copy(cols_h.at[pl.ds(0, SUB)],
                              colb.at[pl.ds(b6 * SUB, SUB)], sem_ld[b6]).wait()
        pltpu.make_async_copy(w_h.at[pl.ds(0, SUB)],
                              wb.at[pl.ds(b6 * SUB, SUB)], sem_ld[b6]).wait()

    def start_gathers(b3, b6):
        @pl.when(c == 0)
        def _():
            for k in range(SUB):
                pltpu.async_copy(table.at[0].at[colb.at[b6 * SUB + k]],
                                 gb.at[pl.ds(b3 * B + k * 128, 128)],
                                 sem_g[b3])

        @pl.when(c == 1)
        def _():
            for k in range(SUB):
                pltpu.async_copy(table.at[1].at[colb.at[b6 * SUB + k]],
                                 gb.at[pl.ds(b3 * B + k * 128, 128)],
                                 sem_g[b3])

    def wait_gathers(b3, b6):
        for k in range(SUB):
            pltpu.make_async_copy(table.at[0].at[colb.at[b6 * SUB + k]],
                                  gb.at[pl.ds(b3 * B + k * 128, 128)],
                                  sem_g[b3]).wait()

    def scale_scatter(b3, b6):
        for k in range(SUB):
            base = b3 * B + k * 128
            wrow = b6 * SUB + k

            def sbody(j16, _, base=base, wrow=wrow):
                wvec = wb[wrow, pl.ds(j16 * 16, 16)]
                r0 = base + j16 * 16
                for l in range(16):
                    gb[r0 + l, :] = gb[r0 + l, :] * wvec[l]
                return 0

            lax.fori_loop(0, 8, sbody, 0)
            pltpu.async_copy(gb.at[pl.ds(base, 128)],
                             acc.at[rowb.at[wrow]], sem_s[b3], add=True)

    def wait_scatters(b3, b6):
        for k in range(SUB):
            pltpu.make_async_copy(gb.at[pl.ds(b3 * B + k * 128, 128)],
                                  acc.at[rowb.at[b6 * SUB + k]],
                                  sem_s[b3]).wait()

    def zb_body(i, _):
        zb[i, :] = jnp.zeros((H,), jnp.float32)
        return 0

    lax.fori_loop(0, ZB, zb_body, 0)

    zbase = s * RPT_A
    nfull = RPT_A // ZB
    for q in range(nfull):
        pltpu.async_copy(zb, acc.at[pl.ds(zbase + q * ZB, ZB)], sz)
    for q in range(nfull):
        pltpu.make_async_copy(zb, acc.at[pl.ds(zbase + q * ZB, ZB)], sz).wait()

    @pl.when(s < NT - 1)
    def _():
        tail = RPT_A - nfull * ZB
        pltpu.sync_copy(zb.at[pl.ds(0, tail)],
                        acc.at[pl.ds(zbase + nfull * ZB, tail)])

    @pl.when(s == NT - 1)
    def _():
        tail = RPT_L - nfull * ZB
        pltpu.sync_copy(zb.at[pl.ds(0, tail)],
                        acc.at[pl.ds(zbase + nfull * ZB, tail)])

    for b in range(3):
        start_loads(b, b)

    plsc.subcore_barrier()

    def group(tg, _):
        for u in range(6):
            g = tg * 6 + u

            @pl.when(g >= 2)
            def _(u=u):
                wait_scatters((u - 2) % 3, (u - 2) % 6)

            @pl.when(g + 3 < C)
            def _(g=g, u=u):
                start_loads(g + 3, (u + 3) % 6)

            wait_loads(u)
            start_gathers(u % 3, u)

            @pl.when(g >= 1)
            def _(u=u):
                wait_gathers((u - 1) % 3, (u - 1) % 6)
                scale_scatter((u - 1) % 3, (u - 1) % 6)
        return 0

    lax.fori_loop(0, C // 6, group, 0)

    wait_gathers((C - 1) % 3, (C - 1) % 6)
    scale_scatter((C - 1) % 3, (C - 1) % 6)
    wait_scatters((C - 2) % 3, (C - 2) % 6)
    wait_scatters((C - 1) % 3, (C - 1) % 6)

    plsc.subcore_barrier()

    return zbase


def _layer_body(table, rows_h, cols_h, w_h, out,
                colb, rowb, wb, gb, zb, acc,
                sl0, sl1, sl2, sl3, sl4, sl5,
                sg0, sg1, sg2, ss0, ss1, ss2, sz):
    c = lax.axis_index("c")
    s = lax.axis_index("s")
    zbase = _edge_phase(c, s, table, rows_h, cols_h, w_h,
                        colb, rowb, wb, gb, zb, acc,
                        (sl0, sl1, sl2, sl3, sl4, sl5),
                        (sg0, sg1, sg2), (ss0, ss1, ss2), sz)

    @pl.when(s < NT - 1)
    def _():
        pltpu.sync_copy(acc.at[pl.ds(zbase, RPT_A)],
                        out.at[c, pl.ds(zbase, RPT_A)])

    @pl.when(s == NT - 1)
    def _():
        pltpu.sync_copy(acc.at[pl.ds(zbase, RPT_L)],
                        out.at[c, pl.ds(zbase, RPT_L)])


def _final_body(table, rows_h, cols_h, w_h, e0h, e1h, e3out, users, items,
                colb, rowb, wb, gb, zb, acc,
                sl0, sl1, sl2, sl3, sl4, sl5,
                sg0, sg1, sg2, ss0, ss1, ss2, sz):
    c = lax.axis_index("c")
    s = lax.axis_index("s")
    zbase = _edge_phase(c, s, table, rows_h, cols_h, w_h,
                        colb, rowb, wb, gb, zb, acc,
                        (sl0, sl1, sl2, sl3, sl4, sl5),
                        (sg0, sg1, sg2), (ss0, ss1, ss2), sz)

    @pl.when(s < NT - 1)
    def _():
        pltpu.sync_copy(acc.at[pl.ds(zbase, RPT_A)],
                        e3out.at[c, pl.ds(zbase, RPT_A)])

    @pl.when(s == NT - 1)
    def _():
        pltpu.sync_copy(acc.at[pl.ds(zbase, RPT_L)],
                        e3out.at[c, pl.ds(zbase, RPT_L)])

    plsc.subcore_barrier()

    sem_m = (sl0, sl1)
    sem_w = (sl2, sl3)
    OFS = (0, 4 * CH)

    def m_load(racc, p):
        o = OFS[p]
        pltpu.async_copy(e0h.at[c, pl.ds(racc, CH)],
                         gb.at[pl.ds(o, CH)], sem_m[p])
        pltpu.async_copy(e1h.at[c, pl.ds(racc, CH)],
                         gb.at[pl.ds(o + CH, CH)], sem_m[p])
        pltpu.async_copy(table.at[c, pl.ds(racc, CH)],
                         gb.at[pl.ds(o + 2 * CH, CH)], sem_m[p])
        pltpu.async_copy(e3out.at[c, pl.ds(racc, CH)],
                         gb.at[pl.ds(o + 3 * CH, CH)], sem_m[p])

    def m_wait_load(p):
        o = OFS[p]
        pltpu.make_async_copy(e0h.at[0, pl.ds(0, CH)],
                              gb.at[pl.ds(o, CH)], sem_m[p]).wait()
        pltpu.make_async_copy(e1h.at[0, pl.ds(0, CH)],
                              gb.at[pl.ds(o + CH, CH)], sem_m[p]).wait()
        pltpu.make_async_copy(table.at[0, pl.ds(0, CH)],
                              gb.at[pl.ds(o + 2 * CH, CH)], sem_m[p]).wait()
        pltpu.make_async_copy(e3out.at[0, pl.ds(0, CH)],
                              gb.at[pl.ds(o + 3 * CH, CH)], sem_m[p]).wait()

    def m_compute(p):
        o = OFS[p]

        def mbody(i, _):
            gb[o + i, :] = (gb[o + i, :] + gb[o + CH + i, :] +
                            gb[o + 2 * CH + i, :] +
                            gb[o + 3 * CH + i, :]) * 0.25
            return 0

        lax.fori_loop(0, CH, mbody, 0)

    def m_store(out_ref, rout, p):
        o = OFS[p]

        @pl.when(c == 0)
        def _():
            pltpu.async_copy(gb.at[pl.ds(o, CH)],
                             out_ref.at[pl.ds(rout, CH), pl.ds(0, H)],
                             sem_w[p])

        @pl.when(c == 1)
        def _():
            pltpu.async_copy(gb.at[pl.ds(o, CH)],
                             out_ref.at[pl.ds(rout, CH), pl.ds(H, H)],
                             sem_w[p])

    def m_wait_store(out_ref, p):
        pltpu.make_async_copy(gb.at[pl.ds(OFS[p], CH)],
                              out_ref.at[pl.ds(0, CH), pl.ds(0, H)],
                              sem_w[p]).wait()

    def run_half(out_ref, node_base, out_base, n2):
        m_load(node_base, 0)

        def mb(i, _):
            ra = node_base + (2 * i) * CH
            ro = out_base + (2 * i) * CH

            @pl.when(i > 0)
            def _():
                m_wait_store(out_ref, 1)

            m_load(ra + CH, 1)
            m_wait_load(0)
            m_compute(0)
            m_store(out_ref, ro, 0)
            m_wait_load(1)
            m_compute(1)
            m_store(out_ref, ro + CH, 1)
            m_wait_store(out_ref, 0)

            @pl.when(i + 1 < n2)
            def _():
                m_load(ra + 2 * CH, 0)
            return 0

        lax.fori_loop(0, n2, mb, 0)
        m_wait_store(out_ref, 1)

    run_half(users, s * UPT, s * UPT, UPT // (2 * CH))
    run_half(items, NU + s * IPT, s * IPT, IPT // (2 * CH))


_MESH = plsc.VectorSubcoreMesh(core_axis_name="c", subcore_axis_name="s")

_SCRATCH = [
    pltpu.VMEM((R6 * SUB, 128), jnp.int32),
    pltpu.VMEM((R6 * SUB, 128), jnp.int32),
    pltpu.VMEM((R6 * SUB, 128), jnp.float32),
    pltpu.VMEM((R3 * B, H), jnp.float32),
    pltpu.VMEM((ZB, H), jnp.float32),
    pltpu.VMEM_SHARED((NN, H), jnp.float32),
] + [pltpu.SemaphoreType.DMA] * 13

_layer = functools.partial(
    pl.kernel,
    out_type=jax.ShapeDtypeStruct((2, NN, H), jnp.float32),
    mesh=_MESH,
    compiler_params=pltpu.CompilerParams(use_tc_tiling_on_sc=False),
    scratch_types=_SCRATCH,
)(_layer_body)

_final = functools.partial(
    pl.kernel,
    out_type=(jax.ShapeDtypeStruct((2, NN, H), jnp.float32),
              jax.ShapeDtypeStruct((NU, D), jnp.float32),
              jax.ShapeDtypeStruct((NI, D), jnp.float32)),
    mesh=_MESH,
    compiler_params=pltpu.CompilerParams(use_tc_tiling_on_sc=False),
    scratch_types=_SCRATCH,
)(_final_body)


def kernel(edge_index, edge_weight, user_emb, item_emb):
    rows = edge_index[0].astype(jnp.int32)
    cols = edge_index[1].astype(jnp.int32)
    w = edge_weight.astype(jnp.float32)

    padr = NR128 - E // 128
    pidx = (jnp.arange(padr * 128, dtype=jnp.int32) % NN).reshape(padr, 128)
    rows_p = jnp.concatenate([rows.reshape(E // 128, 128), pidx], axis=0)
    cols_p = jnp.concatenate([cols.reshape(E // 128, 128), pidx], axis=0)
    w_p = jnp.pad(w.reshape(E // 128, 128), ((0, padr), (0, 0)))

    all_emb = jnp.concatenate([user_emb, item_emb], axis=0)
    e0 = jnp.stack([all_emb[:, :H], all_emb[:, H:]], axis=0)

    e1 = _layer(e0, rows_p, cols_p, w_p)
    e2 = _layer(e1, rows_p, cols_p, w_p)
    _, users, items = _final(e2, rows_p, cols_p, w_p, e0, e1)
    return (users, items)
